# baseline (speedup 1.0000x reference)
import functools

import jax
import jax.numpy as jnp
from jax import lax
from jax.experimental import pallas as pl
from jax.experimental.pallas import tpu as pltpu
from jax.experimental.pallas import tpu_sc as plsc

N_Q = 8
HID = 1024
GROUPS = HID // 16

NW = 32
ROWS_PER_W = 2048
R = 16
NBUF = 2
CH = ROWS_PER_W // R


def _sc_kernel(x_hbm, emb_hbm, out_hbm, emb_v, ib0, ib1, ob0, ob1,
               sin0, sin1, sout0, sout1):
    ibufs = (ib0, ib1)
    obufs = (ob0, ob1)
    sins = (sin0, sin1)
    souts = (sout0, sout1)

    c = lax.axis_index("c")
    s = lax.axis_index("s")
    wid = s * 2 + c
    base = wid * ROWS_PER_W
    q = wid % N_Q

    pltpu.sync_copy(emb_hbm.at[q], emb_v)

    def start_in(b, i):
        pltpu.async_copy(x_hbm.at[pl.ds(base + i * R, R)], ibufs[b], sins[b])

    def wait_in(b):
        pltpu.make_async_copy(x_hbm.at[pl.ds(0, R)], ibufs[b], sins[b]).wait()

    def start_out(b, i):
        pltpu.async_copy(obufs[b], out_hbm.at[pl.ds(base + i * R, R)],
                         souts[b])

    def wait_out(b):
        pltpu.make_async_copy(obufs[b], out_hbm.at[pl.ds(0, R)],
                              souts[b]).wait()

    def compute(b):
        ib = ibufs[b]
        ob = obufs[b]
        for half in range(2):
            g0 = half * (GROUPS // 2)
            embv = [emb_v[pl.ds((g0 + g) * 16, 16)]
                    for g in range(GROUPS // 2)]

            def row_body(r, _):
                for g in range(GROUPS // 2):
                    sl = pl.ds((g0 + g) * 16, 16)
                    ob[r, sl] = ib[r, sl] + embv[g]
                return 0

            lax.fori_loop(0, R, row_body, 0)

    for b in range(NBUF):
        start_in(b, b)

    def outer(k, _):
        for b in range(NBUF):
            i = k * NBUF + b
            wait_in(b)

            @pl.when(i >= NBUF)
            def _():
                wait_out(b)

            compute(b)
            start_out(b, i)

            @pl.when(i + NBUF < CH)
            def _():
                start_in(b, i + NBUF)

        return 0

    lax.fori_loop(0, CH // NBUF, outer, 0)

    for b in range(NBUF):
        wait_out(b)


def kernel(x, emb_table):
    b, q, t, h = x.shape
    xf = x.reshape(b * q * t, h)

    mesh = plsc.VectorSubcoreMesh(core_axis_name="c", subcore_axis_name="s")
    run = pl.kernel(
        _sc_kernel,
        mesh=mesh,
        out_type=jax.ShapeDtypeStruct((b * q * t, h), x.dtype),
        scratch_types=[
            pltpu.VMEM((h,), jnp.float32),
            pltpu.VMEM((R, h), jnp.float32),
            pltpu.VMEM((R, h), jnp.float32),
            pltpu.VMEM((R, h), jnp.float32),
            pltpu.VMEM((R, h), jnp.float32),
            pltpu.SemaphoreType.DMA,
            pltpu.SemaphoreType.DMA,
            pltpu.SemaphoreType.DMA,
            pltpu.SemaphoreType.DMA,
        ],
    )
    out = run(xf, emb_table)
    return out.reshape(b, q, t, h)
